# baseline (speedup 1.0000x reference)
import functools

import jax
import jax.numpy as jnp
from jax import lax
from jax.experimental import pallas as pl
from jax.experimental.pallas import tpu as pltpu
from jax.experimental.pallas import tpu_sc as plsc

N_ROWS = 16384
DIM = 256
N_CODES = 8192
COMMIT = 0.25

BM = 256
BN = 256
N_RT = N_ROWS // BM
N_CT = N_CODES // BN

BIG_I32 = 2**30

_WIN_START = (0, 11, 22)
_WIN_END = (10, 21, 31)


def _bf16(v):
    return v.astype(jnp.bfloat16).astype(jnp.float32)


def _dist_body(x_ref, e_ref, xsq_ref, esq_ref, idx_out, min_out,
               dist_scr, win_v, win_i, glob_v, glob_i, glob_e):
    j = pl.program_id(1)

    x = x_ref[...]
    e = e_ref[...]
    mm = jnp.dot(x, e, preferred_element_type=jnp.float32)
    dist_scr[...] = (xsq_ref[...] + esq_ref[...]) - 2.0 * mm
    dist = dist_scr[...]

    local_min = jnp.min(dist, axis=1, keepdims=True)
    col = jax.lax.broadcasted_iota(jnp.int32, (BM, BN), 1) + j * BN
    local_arg = jnp.min(jnp.where(dist == local_min, col, BIG_I32),
                        axis=1, keepdims=True)

    is_start = (j == _WIN_START[0]) | (j == _WIN_START[1]) | (j == _WIN_START[2])

    @pl.when(is_start)
    def _win_init():
        win_v[...] = local_min
        win_i[...] = local_arg

    @pl.when(jnp.logical_not(is_start))
    def _win_merge():
        better = local_min < win_v[...]
        win_v[...] = jnp.where(better, local_min, win_v[...])
        win_i[...] = jnp.where(better, local_arg, win_i[...])

    @pl.when(j == _WIN_END[0])
    def _glob_init():
        glob_v[...] = _bf16(win_v[...])
        glob_i[...] = win_i[...]
        glob_e[...] = win_v[...]

    @pl.when((j == _WIN_END[1]) | (j == _WIN_END[2]))
    def _glob_merge():
        gv, gi = glob_v[...], glob_i[...]
        wv, wi = win_v[...], win_i[...]
        keep = (gv < wv) | ((gv == wv) & (gi < wi))
        glob_i[...] = jnp.where(keep, gi, wi)
        glob_e[...] = jnp.where(keep, glob_e[...], wv)
        glob_v[...] = _bf16(jnp.where(keep, gv, wv))

    @pl.when(j == N_CT - 1)
    def _emit():
        idx_out[...] = glob_i[...]
        min_out[...] = glob_e[...]


def _argmin_tc(flat, embed, x_sq, e_sq):
    return pl.pallas_call(
        _dist_body,
        grid=(N_RT, N_CT),
        in_specs=[
            pl.BlockSpec((BM, DIM), lambda i, j: (i, 0)),
            pl.BlockSpec((DIM, BN), lambda i, j: (0, j)),
            pl.BlockSpec((BM, 1), lambda i, j: (i, 0)),
            pl.BlockSpec((1, BN), lambda i, j: (0, j)),
        ],
        out_specs=[
            pl.BlockSpec((BM, 1), lambda i, j: (i, 0)),
            pl.BlockSpec((BM, 1), lambda i, j: (i, 0)),
        ],
        out_shape=[
            jax.ShapeDtypeStruct((N_ROWS, 1), jnp.int32),
            jax.ShapeDtypeStruct((N_ROWS, 1), jnp.float32),
        ],
        scratch_shapes=[
            pltpu.VMEM((BM, BN), jnp.float32),
            pltpu.VMEM((BM, 1), jnp.float32),
            pltpu.VMEM((BM, 1), jnp.int32),
            pltpu.VMEM((BM, 1), jnp.float32),
            pltpu.VMEM((BM, 1), jnp.int32),
            pltpu.VMEM((BM, 1), jnp.float32),
        ],
    )(flat, embed, x_sq, e_sq)


_NC = 2
_NS = 16
_NW = _NC * _NS
_BPW = N_ROWS // _NW
_CHUNK = 128
_NCH = _BPW // _CHUNK
_HSLICE = N_CODES // _NS


def _sc_gather_hist(idx2, table):
    mesh = plsc.VectorSubcoreMesh(core_axis_name="c", subcore_axis_name="s")

    @functools.partial(
        pl.kernel,
        out_type=[
            jax.ShapeDtypeStruct((N_ROWS, DIM), jnp.float32),
            jax.ShapeDtypeStruct((_NC, N_CODES), jnp.float32),
        ],
        mesh=mesh,
        scratch_types=[
            pltpu.VMEM((_NCH, _CHUNK), jnp.int32),
            pltpu.VMEM((_CHUNK, DIM), jnp.float32),
            pltpu.VMEM((_CHUNK,), jnp.float32),
            pltpu.VMEM((_HSLICE,), jnp.float32),
            pltpu.VMEM_SHARED((N_CODES,), jnp.float32),
            pltpu.SemaphoreType.DMA,
        ],
    )
    def k(idx_hbm, table_hbm, out_hbm, hist_hbm,
          idx_v, rows_v, ones_v, stage_v, hist_sh, sem):
        cid = lax.axis_index("c")
        sid = lax.axis_index("s")
        wid = sid * _NC + cid
        base = wid * _BPW

        pltpu.sync_copy(idx_hbm.at[wid], idx_v)

        def fill(r, val, n):
            def body(t, _):
                r[pl.ds(t * 16, 16)] = jnp.full((16,), val, jnp.float32)
                return 0
            lax.fori_loop(0, n // 16, body, 0)

        fill(ones_v, 1.0, _CHUNK)
        fill(stage_v, 0.0, _HSLICE)
        pltpu.sync_copy(stage_v, hist_sh.at[pl.ds(sid * _HSLICE, _HSLICE)])
        plsc.subcore_barrier()

        def chunk(c, _):
            pltpu.async_copy(table_hbm.at[idx_v.at[c]], rows_v, sem).wait()
            pltpu.sync_copy(rows_v,
                            out_hbm.at[pl.ds(base + c * _CHUNK, _CHUNK)])
            pltpu.sync_copy(ones_v, hist_sh.at[idx_v.at[c]], add=True)
            return 0
        lax.fori_loop(0, _NCH, chunk, 0)

        plsc.subcore_barrier()
        pltpu.sync_copy(hist_sh.at[pl.ds(sid * _HSLICE, _HSLICE)], stage_v)
        pltpu.sync_copy(stage_v,
                        hist_hbm.at[cid, pl.ds(sid * _HSLICE, _HSLICE)])

    return k(idx2, table)


def _finalize_body(min_ref, hist_ref, loss_ref, perp_ref):
    loss_ref[0, 0] = (COMMIT / (N_ROWS * DIM)) * jnp.sum(min_ref[...])
    counts = hist_ref[0:64, :] + hist_ref[64:128, :]
    p = counts * (1.0 / N_ROWS)
    ent = -jnp.sum(p * jnp.log(p + 1e-10))
    perp_ref[0, 0] = jnp.exp(ent)


def _finalize_tc(min2d, hist2d):
    return pl.pallas_call(
        _finalize_body,
        out_shape=[
            jax.ShapeDtypeStruct((1, 1), jnp.float32),
            jax.ShapeDtypeStruct((1, 1), jnp.float32),
        ],
        out_specs=[
            pl.BlockSpec(memory_space=pltpu.SMEM),
            pl.BlockSpec(memory_space=pltpu.SMEM),
        ],
    )(min2d, hist2d)


def kernel(inputs, embed):
    B, C, H, W = inputs.shape
    flat = jnp.transpose(inputs, (0, 2, 3, 1)).reshape(-1, C)
    table = embed.T

    x_sq = jnp.sum(flat ** 2, axis=1, keepdims=True)
    e_sq = jnp.sum(embed ** 2, axis=0).reshape(1, N_CODES)

    idx_col, min_col = _argmin_tc(flat, embed, x_sq, e_sq)
    idx_flat = idx_col.reshape(-1)

    idx2 = idx_flat.reshape(_NW, _NCH, _CHUNK)
    quant_flat, hist = _sc_gather_hist(idx2, table)

    loss2d, perp2d = _finalize_tc(min_col.reshape(128, 128),
                                  hist.reshape(128, 128))

    quantized_out = jnp.transpose(quant_flat.reshape(B, H, W, C), (0, 3, 1, 2))
    return (quantized_out,
            loss2d.reshape(()),
            idx_flat,
            perp2d.reshape(()))


# speedup vs baseline: 1.1273x; 1.1273x over previous
import functools

import jax
import jax.numpy as jnp
from jax import lax
from jax.experimental import pallas as pl
from jax.experimental.pallas import tpu as pltpu
from jax.experimental.pallas import tpu_sc as plsc

N_ROWS = 16384
DIM = 256
N_CODES = 8192
COMMIT = 0.25

BM = 256
BN = 256
N_RT = N_ROWS // BM
N_CT = N_CODES // BN

BIG_I32 = 2**30

_WIN_START = (0, 11, 22)
_WIN_END = (10, 21, 31)


def _bf16(v):
    return v.astype(jnp.bfloat16).astype(jnp.float32)


def _dist_body(et_ref, xt_ref, xsq_ref, esq_ref, idx_out, min_out,
               win_v, win_i, glob_v, glob_i, glob_e):
    j = pl.program_id(1)

    et = et_ref[...]
    xt = xt_ref[...]
    mm = jnp.dot(et, xt, preferred_element_type=jnp.float32)
    dist = (xsq_ref[...] + esq_ref[...]) - 2.0 * mm

    local_min = jnp.min(dist, axis=0, keepdims=True)
    row = jax.lax.broadcasted_iota(jnp.int32, (BN, BM), 0) + j * BN
    local_arg = jnp.min(jnp.where(dist == local_min, row, BIG_I32),
                        axis=0, keepdims=True)

    is_start = (j == _WIN_START[0]) | (j == _WIN_START[1]) | (j == _WIN_START[2])

    @pl.when(is_start)
    def _win_init():
        win_v[...] = local_min
        win_i[...] = local_arg

    @pl.when(jnp.logical_not(is_start))
    def _win_merge():
        better = local_min < win_v[...]
        win_v[...] = jnp.where(better, local_min, win_v[...])
        win_i[...] = jnp.where(better, local_arg, win_i[...])

    @pl.when(j == _WIN_END[0])
    def _glob_init():
        glob_v[...] = _bf16(win_v[...])
        glob_i[...] = win_i[...]
        glob_e[...] = win_v[...]

    @pl.when((j == _WIN_END[1]) | (j == _WIN_END[2]))
    def _glob_merge():
        gv, gi = glob_v[...], glob_i[...]
        wv, wi = win_v[...], win_i[...]
        keep = (gv < wv) | ((gv == wv) & (gi < wi))
        glob_i[...] = jnp.where(keep, gi, wi)
        glob_e[...] = jnp.where(keep, glob_e[...], wv)
        glob_v[...] = _bf16(jnp.where(keep, gv, wv))

    @pl.when(j == N_CT - 1)
    def _emit():
        idx_out[...] = glob_i[...].reshape(1, 1, BM)
        min_out[...] = glob_e[...].reshape(1, 1, BM)


def _argmin_tc(table, flat_t, x_sq_t, e_sq_col):
    return pl.pallas_call(
        _dist_body,
        grid=(N_RT, N_CT),
        in_specs=[
            pl.BlockSpec((BN, DIM), lambda i, j: (j, 0)),
            pl.BlockSpec((DIM, BM), lambda i, j: (0, i)),
            pl.BlockSpec((1, BM), lambda i, j: (0, i)),
            pl.BlockSpec((BN, 1), lambda i, j: (j, 0)),
        ],
        out_specs=[
            pl.BlockSpec((1, 1, BM), lambda i, j: (i, 0, 0)),
            pl.BlockSpec((1, 1, BM), lambda i, j: (i, 0, 0)),
        ],
        out_shape=[
            jax.ShapeDtypeStruct((N_RT, 1, BM), jnp.int32),
            jax.ShapeDtypeStruct((N_RT, 1, BM), jnp.float32),
        ],
        scratch_shapes=[
            pltpu.VMEM((1, BM), jnp.float32),
            pltpu.VMEM((1, BM), jnp.int32),
            pltpu.VMEM((1, BM), jnp.float32),
            pltpu.VMEM((1, BM), jnp.int32),
            pltpu.VMEM((1, BM), jnp.float32),
        ],
    )(table, flat_t, x_sq_t, e_sq_col)


_NC = 2
_NS = 16
_NW = _NC * _NS
_BPW = N_ROWS // _NW
_CHUNK = 128
_NCH = _BPW // _CHUNK
_HSLICE = N_CODES // _NS


def _sc_gather_hist(idx2, table):
    mesh = plsc.VectorSubcoreMesh(core_axis_name="c", subcore_axis_name="s")

    @functools.partial(
        pl.kernel,
        out_type=[
            jax.ShapeDtypeStruct((N_ROWS, DIM), jnp.float32),
            jax.ShapeDtypeStruct((_NC, N_CODES), jnp.float32),
        ],
        mesh=mesh,
        scratch_types=[
            pltpu.VMEM((_NCH, _CHUNK), jnp.int32),
            pltpu.VMEM((_CHUNK, DIM), jnp.float32),
            pltpu.VMEM((_CHUNK,), jnp.float32),
            pltpu.VMEM((_HSLICE,), jnp.float32),
            pltpu.VMEM_SHARED((N_CODES,), jnp.float32),
            pltpu.SemaphoreType.DMA,
        ],
    )
    def k(idx_hbm, table_hbm, out_hbm, hist_hbm,
          idx_v, rows_v, ones_v, stage_v, hist_sh, sem):
        cid = lax.axis_index("c")
        sid = lax.axis_index("s")
        wid = sid * _NC + cid
        base = wid * _BPW

        pltpu.sync_copy(idx_hbm.at[wid], idx_v)

        def fill(r, val, n):
            def body(t, _):
                r[pl.ds(t * 16, 16)] = jnp.full((16,), val, jnp.float32)
                return 0
            lax.fori_loop(0, n // 16, body, 0)

        fill(ones_v, 1.0, _CHUNK)
        fill(stage_v, 0.0, _HSLICE)
        pltpu.sync_copy(stage_v, hist_sh.at[pl.ds(sid * _HSLICE, _HSLICE)])
        plsc.subcore_barrier()

        def chunk(c, _):
            pltpu.async_copy(table_hbm.at[idx_v.at[c]], rows_v, sem).wait()
            pltpu.sync_copy(rows_v,
                            out_hbm.at[pl.ds(base + c * _CHUNK, _CHUNK)])
            pltpu.sync_copy(ones_v, hist_sh.at[idx_v.at[c]], add=True)
            return 0
        lax.fori_loop(0, _NCH, chunk, 0)

        plsc.subcore_barrier()
        pltpu.sync_copy(hist_sh.at[pl.ds(sid * _HSLICE, _HSLICE)], stage_v)
        pltpu.sync_copy(stage_v,
                        hist_hbm.at[cid, pl.ds(sid * _HSLICE, _HSLICE)])

    return k(idx2, table)


def _finalize_body(min_ref, hist_ref, loss_ref, perp_ref):
    loss_ref[0, 0] = (COMMIT / (N_ROWS * DIM)) * jnp.sum(min_ref[...])
    counts = hist_ref[0:64, :] + hist_ref[64:128, :]
    p = counts * (1.0 / N_ROWS)
    ent = -jnp.sum(p * jnp.log(p + 1e-10))
    perp_ref[0, 0] = jnp.exp(ent)


def _finalize_tc(min2d, hist2d):
    return pl.pallas_call(
        _finalize_body,
        out_shape=[
            jax.ShapeDtypeStruct((1, 1), jnp.float32),
            jax.ShapeDtypeStruct((1, 1), jnp.float32),
        ],
        out_specs=[
            pl.BlockSpec(memory_space=pltpu.SMEM),
            pl.BlockSpec(memory_space=pltpu.SMEM),
        ],
    )(min2d, hist2d)


def kernel(inputs, embed):
    B, C, H, W = inputs.shape
    flat = jnp.transpose(inputs, (0, 2, 3, 1)).reshape(-1, C)
    table = embed.T

    x_sq_t = jnp.sum(flat ** 2, axis=1, keepdims=True).reshape(1, N_ROWS)
    e_sq_col = jnp.sum(embed ** 2, axis=0).reshape(N_CODES, 1)
    flat_t = jnp.transpose(inputs, (1, 0, 2, 3)).reshape(C, -1)

    idx_col, min_col = _argmin_tc(table, flat_t, x_sq_t, e_sq_col)
    idx_flat = idx_col.reshape(-1)

    idx2 = idx_flat.reshape(_NW, _NCH, _CHUNK)
    quant_flat, hist = _sc_gather_hist(idx2, table)

    loss2d, perp2d = _finalize_tc(min_col.reshape(128, 128),
                                  hist.reshape(128, 128))

    quantized_out = jnp.transpose(quant_flat.reshape(B, H, W, C), (0, 3, 1, 2))
    return (quantized_out,
            loss2d.reshape(()),
            idx_flat,
            perp2d.reshape(()))


# speedup vs baseline: 3.2621x; 2.8938x over previous
import functools

import jax
import jax.numpy as jnp
from jax import lax
from jax.experimental import pallas as pl
from jax.experimental.pallas import tpu as pltpu
from jax.experimental.pallas import tpu_sc as plsc

N_ROWS = 16384
DIM = 256
N_CODES = 8192
COMMIT = 0.25

BM = 1024
BN = 256
N_RT = N_ROWS // BM
N_CT = N_CODES // BN

BIG_I32 = 2**30

_WIN_START = (0, 11, 22)
_WIN_END = (10, 21, 31)


def _bf16(v):
    return v.astype(jnp.bfloat16).astype(jnp.float32)


def _dist_body(et_ref, xt_ref, xsq_ref, esq_ref, idx_out, min_out,
               win_v, win_i, glob_v, glob_i, glob_e):
    j = pl.program_id(1)

    et = et_ref[...]
    xt = xt_ref[...]
    mm = jnp.dot(et, xt, preferred_element_type=jnp.float32)
    dist = (xsq_ref[...] + esq_ref[...]) - 2.0 * mm

    local_min = jnp.min(dist, axis=0, keepdims=True)
    row = jax.lax.broadcasted_iota(jnp.int32, (BN, BM), 0) + j * BN
    local_arg = jnp.min(jnp.where(dist == local_min, row, BIG_I32),
                        axis=0, keepdims=True)

    is_start = (j == _WIN_START[0]) | (j == _WIN_START[1]) | (j == _WIN_START[2])

    @pl.when(is_start)
    def _win_init():
        win_v[...] = local_min
        win_i[...] = local_arg

    @pl.when(jnp.logical_not(is_start))
    def _win_merge():
        better = local_min < win_v[...]
        win_v[...] = jnp.where(better, local_min, win_v[...])
        win_i[...] = jnp.where(better, local_arg, win_i[...])

    @pl.when(j == _WIN_END[0])
    def _glob_init():
        glob_v[...] = _bf16(win_v[...])
        glob_i[...] = win_i[...]
        glob_e[...] = win_v[...]

    @pl.when((j == _WIN_END[1]) | (j == _WIN_END[2]))
    def _glob_merge():
        gv, gi = glob_v[...], glob_i[...]
        wv, wi = win_v[...], win_i[...]
        keep = (gv < wv) | ((gv == wv) & (gi < wi))
        glob_i[...] = jnp.where(keep, gi, wi)
        glob_e[...] = jnp.where(keep, glob_e[...], wv)
        glob_v[...] = _bf16(jnp.where(keep, gv, wv))

    @pl.when(j == N_CT - 1)
    def _emit():
        idx_out[...] = glob_i[...].reshape(1, 1, BM)
        min_out[...] = glob_e[...].reshape(1, 1, BM)


def _argmin_tc(table, flat_t, x_sq_t, e_sq_col):
    return pl.pallas_call(
        _dist_body,
        grid=(N_RT, N_CT),
        in_specs=[
            pl.BlockSpec((BN, DIM), lambda i, j: (j, 0)),
            pl.BlockSpec((DIM, BM), lambda i, j: (0, i)),
            pl.BlockSpec((1, BM), lambda i, j: (0, i)),
            pl.BlockSpec((BN, 1), lambda i, j: (j, 0)),
        ],
        out_specs=[
            pl.BlockSpec((1, 1, BM), lambda i, j: (i, 0, 0)),
            pl.BlockSpec((1, 1, BM), lambda i, j: (i, 0, 0)),
        ],
        out_shape=[
            jax.ShapeDtypeStruct((N_RT, 1, BM), jnp.int32),
            jax.ShapeDtypeStruct((N_RT, 1, BM), jnp.float32),
        ],
        scratch_shapes=[
            pltpu.VMEM((1, BM), jnp.float32),
            pltpu.VMEM((1, BM), jnp.int32),
            pltpu.VMEM((1, BM), jnp.float32),
            pltpu.VMEM((1, BM), jnp.int32),
            pltpu.VMEM((1, BM), jnp.float32),
        ],
    )(table, flat_t, x_sq_t, e_sq_col)


_NC = 2
_NS = 16
_NW = _NC * _NS
_BPW = N_ROWS // _NW
_CHUNK = 128
_NCH = _BPW // _CHUNK
_HSLICE = N_CODES // _NS


def _sc_gather_hist(idx2, table):
    mesh = plsc.VectorSubcoreMesh(core_axis_name="c", subcore_axis_name="s")

    @functools.partial(
        pl.kernel,
        out_type=[
            jax.ShapeDtypeStruct((N_ROWS, DIM), jnp.float32),
            jax.ShapeDtypeStruct((_NC, N_CODES), jnp.float32),
        ],
        mesh=mesh,
        scratch_types=[
            pltpu.VMEM((_NCH, _CHUNK), jnp.int32),
            pltpu.VMEM((_CHUNK, DIM), jnp.float32),
            pltpu.VMEM((_CHUNK,), jnp.float32),
            pltpu.VMEM((_HSLICE,), jnp.float32),
            pltpu.VMEM_SHARED((N_CODES,), jnp.float32),
            pltpu.SemaphoreType.DMA,
        ],
    )
    def k(idx_hbm, table_hbm, out_hbm, hist_hbm,
          idx_v, rows_v, ones_v, stage_v, hist_sh, sem):
        cid = lax.axis_index("c")
        sid = lax.axis_index("s")
        wid = sid * _NC + cid
        base = wid * _BPW

        pltpu.sync_copy(idx_hbm.at[wid], idx_v)

        def fill(r, val, n):
            def body(t, _):
                r[pl.ds(t * 16, 16)] = jnp.full((16,), val, jnp.float32)
                return 0
            lax.fori_loop(0, n // 16, body, 0)

        fill(ones_v, 1.0, _CHUNK)
        fill(stage_v, 0.0, _HSLICE)
        pltpu.sync_copy(stage_v, hist_sh.at[pl.ds(sid * _HSLICE, _HSLICE)])
        plsc.subcore_barrier()

        def chunk(c, _):
            pltpu.async_copy(table_hbm.at[idx_v.at[c]], rows_v, sem).wait()
            pltpu.sync_copy(rows_v,
                            out_hbm.at[pl.ds(base + c * _CHUNK, _CHUNK)])
            pltpu.sync_copy(ones_v, hist_sh.at[idx_v.at[c]], add=True)
            return 0
        lax.fori_loop(0, _NCH, chunk, 0)

        plsc.subcore_barrier()
        pltpu.sync_copy(hist_sh.at[pl.ds(sid * _HSLICE, _HSLICE)], stage_v)
        pltpu.sync_copy(stage_v,
                        hist_hbm.at[cid, pl.ds(sid * _HSLICE, _HSLICE)])

    return k(idx2, table)


def _finalize_body(min_ref, hist_ref, loss_ref, perp_ref):
    loss_ref[0, 0] = (COMMIT / (N_ROWS * DIM)) * jnp.sum(min_ref[...])
    counts = hist_ref[0:64, :] + hist_ref[64:128, :]
    p = counts * (1.0 / N_ROWS)
    ent = -jnp.sum(p * jnp.log(p + 1e-10))
    perp_ref[0, 0] = jnp.exp(ent)


def _finalize_tc(min2d, hist2d):
    return pl.pallas_call(
        _finalize_body,
        out_shape=[
            jax.ShapeDtypeStruct((1, 1), jnp.float32),
            jax.ShapeDtypeStruct((1, 1), jnp.float32),
        ],
        out_specs=[
            pl.BlockSpec(memory_space=pltpu.SMEM),
            pl.BlockSpec(memory_space=pltpu.SMEM),
        ],
    )(min2d, hist2d)


def kernel(inputs, embed):
    B, C, H, W = inputs.shape
    flat = jnp.transpose(inputs, (0, 2, 3, 1)).reshape(-1, C)
    table = embed.T

    x_sq_t = jnp.sum(flat ** 2, axis=1, keepdims=True).reshape(1, N_ROWS)
    e_sq_col = jnp.sum(embed ** 2, axis=0).reshape(N_CODES, 1)
    flat_t = jnp.transpose(inputs, (1, 0, 2, 3)).reshape(C, -1)

    idx_col, min_col = _argmin_tc(table, flat_t, x_sq_t, e_sq_col)
    idx_flat = idx_col.reshape(-1)

    idx2 = idx_flat.reshape(_NW, _NCH, _CHUNK)
    quant_flat, hist = _sc_gather_hist(idx2, table)

    loss2d, perp2d = _finalize_tc(min_col.reshape(128, 128),
                                  hist.reshape(128, 128))

    quantized_out = jnp.transpose(quant_flat.reshape(B, H, W, C), (0, 3, 1, 2))
    return (quantized_out,
            loss2d.reshape(()),
            idx_flat,
            perp2d.reshape(()))


# speedup vs baseline: 4.5588x; 1.3975x over previous
import functools

import jax
import jax.numpy as jnp
from jax import lax
from jax.experimental import pallas as pl
from jax.experimental.pallas import tpu as pltpu
from jax.experimental.pallas import tpu_sc as plsc

N_ROWS = 16384
DIM = 256
N_CODES = 8192
COMMIT = 0.25

BM = 2048
BN = 256
N_RT = N_ROWS // BM
N_CT = N_CODES // BN

BIG_I32 = 2**30

_WIN_START = (0, 11, 22)
_WIN_END = (10, 21, 31)


def _bf16(v):
    return v.astype(jnp.bfloat16).astype(jnp.float32)


def _dist_body(et_ref, xt_ref, xsq_ref, esq_ref, idx_out, min_out,
               win_v, win_i, glob_v, glob_i, glob_e):
    j = pl.program_id(1)

    et = et_ref[...]
    xt = xt_ref[...]
    mm = jnp.dot(et, xt, preferred_element_type=jnp.float32)
    dist = (xsq_ref[...] + esq_ref[...]) - 2.0 * mm

    local_min = jnp.min(dist, axis=0, keepdims=True)
    row = jax.lax.broadcasted_iota(jnp.int32, (BN, BM), 0) + j * BN
    local_arg = jnp.min(jnp.where(dist == local_min, row, BIG_I32),
                        axis=0, keepdims=True)

    is_start = (j == _WIN_START[0]) | (j == _WIN_START[1]) | (j == _WIN_START[2])

    @pl.when(is_start)
    def _win_init():
        win_v[...] = local_min
        win_i[...] = local_arg

    @pl.when(jnp.logical_not(is_start))
    def _win_merge():
        better = local_min < win_v[...]
        win_v[...] = jnp.where(better, local_min, win_v[...])
        win_i[...] = jnp.where(better, local_arg, win_i[...])

    @pl.when(j == _WIN_END[0])
    def _glob_init():
        glob_v[...] = _bf16(win_v[...])
        glob_i[...] = win_i[...]
        glob_e[...] = win_v[...]

    @pl.when((j == _WIN_END[1]) | (j == _WIN_END[2]))
    def _glob_merge():
        gv, gi = glob_v[...], glob_i[...]
        wv, wi = win_v[...], win_i[...]
        keep = (gv < wv) | ((gv == wv) & (gi < wi))
        glob_i[...] = jnp.where(keep, gi, wi)
        glob_e[...] = jnp.where(keep, glob_e[...], wv)
        glob_v[...] = _bf16(jnp.where(keep, gv, wv))

    @pl.when(j == N_CT - 1)
    def _emit():
        idx_out[...] = glob_i[...].reshape(1, 1, BM)
        min_out[...] = glob_e[...].reshape(1, 1, BM)


def _argmin_tc(table, flat_t, x_sq_t, e_sq_col):
    return pl.pallas_call(
        _dist_body,
        grid=(N_RT, N_CT),
        in_specs=[
            pl.BlockSpec((BN, DIM), lambda i, j: (j, 0)),
            pl.BlockSpec((DIM, BM), lambda i, j: (0, i)),
            pl.BlockSpec((1, BM), lambda i, j: (0, i)),
            pl.BlockSpec((BN, 1), lambda i, j: (j, 0)),
        ],
        out_specs=[
            pl.BlockSpec((1, 1, BM), lambda i, j: (i, 0, 0)),
            pl.BlockSpec((1, 1, BM), lambda i, j: (i, 0, 0)),
        ],
        out_shape=[
            jax.ShapeDtypeStruct((N_RT, 1, BM), jnp.int32),
            jax.ShapeDtypeStruct((N_RT, 1, BM), jnp.float32),
        ],
        scratch_shapes=[
            pltpu.VMEM((1, BM), jnp.float32),
            pltpu.VMEM((1, BM), jnp.int32),
            pltpu.VMEM((1, BM), jnp.float32),
            pltpu.VMEM((1, BM), jnp.int32),
            pltpu.VMEM((1, BM), jnp.float32),
        ],
    )(table, flat_t, x_sq_t, e_sq_col)


_NC = 2
_NS = 16
_NW = _NC * _NS
_BPW = N_ROWS // _NW
_CHUNK = 128
_NCH = _BPW // _CHUNK
_HSLICE = N_CODES // _NS


def _sc_gather_hist(idx2, table):
    mesh = plsc.VectorSubcoreMesh(core_axis_name="c", subcore_axis_name="s")

    @functools.partial(
        pl.kernel,
        out_type=[
            jax.ShapeDtypeStruct((N_ROWS, DIM), jnp.float32),
            jax.ShapeDtypeStruct((_NC, N_CODES), jnp.float32),
        ],
        mesh=mesh,
        scratch_types=[
            pltpu.VMEM((_NCH, _CHUNK), jnp.int32),
            pltpu.VMEM((_CHUNK, DIM), jnp.float32),
            pltpu.VMEM((_CHUNK,), jnp.float32),
            pltpu.VMEM((_HSLICE,), jnp.float32),
            pltpu.VMEM_SHARED((N_CODES,), jnp.float32),
            pltpu.SemaphoreType.DMA,
        ],
    )
    def k(idx_hbm, table_hbm, out_hbm, hist_hbm,
          idx_v, rows_v, ones_v, stage_v, hist_sh, sem):
        cid = lax.axis_index("c")
        sid = lax.axis_index("s")
        wid = sid * _NC + cid
        base = wid * _BPW

        pltpu.sync_copy(idx_hbm.at[wid], idx_v)

        def fill(r, val, n):
            def body(t, _):
                r[pl.ds(t * 16, 16)] = jnp.full((16,), val, jnp.float32)
                return 0
            lax.fori_loop(0, n // 16, body, 0)

        fill(ones_v, 1.0, _CHUNK)
        fill(stage_v, 0.0, _HSLICE)
        pltpu.sync_copy(stage_v, hist_sh.at[pl.ds(sid * _HSLICE, _HSLICE)])
        plsc.subcore_barrier()

        def chunk(c, _):
            pltpu.async_copy(table_hbm.at[idx_v.at[c]], rows_v, sem).wait()
            pltpu.sync_copy(rows_v,
                            out_hbm.at[pl.ds(base + c * _CHUNK, _CHUNK)])
            pltpu.sync_copy(ones_v, hist_sh.at[idx_v.at[c]], add=True)
            return 0
        lax.fori_loop(0, _NCH, chunk, 0)

        plsc.subcore_barrier()
        pltpu.sync_copy(hist_sh.at[pl.ds(sid * _HSLICE, _HSLICE)], stage_v)
        pltpu.sync_copy(stage_v,
                        hist_hbm.at[cid, pl.ds(sid * _HSLICE, _HSLICE)])

    return k(idx2, table)


def _finalize_body(min_ref, hist_ref, loss_ref, perp_ref):
    loss_ref[0, 0] = (COMMIT / (N_ROWS * DIM)) * jnp.sum(min_ref[...])
    counts = hist_ref[0:64, :] + hist_ref[64:128, :]
    p = counts * (1.0 / N_ROWS)
    ent = -jnp.sum(p * jnp.log(p + 1e-10))
    perp_ref[0, 0] = jnp.exp(ent)


def _finalize_tc(min2d, hist2d):
    return pl.pallas_call(
        _finalize_body,
        out_shape=[
            jax.ShapeDtypeStruct((1, 1), jnp.float32),
            jax.ShapeDtypeStruct((1, 1), jnp.float32),
        ],
        out_specs=[
            pl.BlockSpec(memory_space=pltpu.SMEM),
            pl.BlockSpec(memory_space=pltpu.SMEM),
        ],
    )(min2d, hist2d)


def kernel(inputs, embed):
    B, C, H, W = inputs.shape
    flat = jnp.transpose(inputs, (0, 2, 3, 1)).reshape(-1, C)
    table = embed.T

    x_sq_t = jnp.sum(flat ** 2, axis=1, keepdims=True).reshape(1, N_ROWS)
    e_sq_col = jnp.sum(embed ** 2, axis=0).reshape(N_CODES, 1)
    flat_t = jnp.transpose(inputs, (1, 0, 2, 3)).reshape(C, -1)

    idx_col, min_col = _argmin_tc(table, flat_t, x_sq_t, e_sq_col)
    idx_flat = idx_col.reshape(-1)

    idx2 = idx_flat.reshape(_NW, _NCH, _CHUNK)
    quant_flat, hist = _sc_gather_hist(idx2, table)

    loss2d, perp2d = _finalize_tc(min_col.reshape(128, 128),
                                  hist.reshape(128, 128))

    quantized_out = jnp.transpose(quant_flat.reshape(B, H, W, C), (0, 3, 1, 2))
    return (quantized_out,
            loss2d.reshape(()),
            idx_flat,
            perp2d.reshape(()))


# speedup vs baseline: 4.9711x; 1.0904x over previous
import functools

import jax
import jax.numpy as jnp
from jax import lax
from jax.experimental import pallas as pl
from jax.experimental.pallas import tpu as pltpu
from jax.experimental.pallas import tpu_sc as plsc

N_ROWS = 16384
DIM = 256
N_CODES = 8192
COMMIT = 0.25

BM = 4096
BN = 256
N_RT = N_ROWS // BM
N_CT = N_CODES // BN

BIG_I32 = 2**30

_WIN_START = (0, 11, 22)
_WIN_END = (10, 21, 31)


def _bf16(v):
    return v.astype(jnp.bfloat16).astype(jnp.float32)


def _dist_body(et_ref, xt_ref, xsq_ref, esq_ref, idx_out, min_out,
               win_v, win_i, glob_v, glob_i, glob_e):
    j = pl.program_id(1)

    et = et_ref[...]
    xt = xt_ref[...]
    mm = jnp.dot(et, xt, preferred_element_type=jnp.float32)
    dist = (xsq_ref[...] + esq_ref[...]) - 2.0 * mm

    local_min = jnp.min(dist, axis=0, keepdims=True)
    row = jax.lax.broadcasted_iota(jnp.int32, (BN, BM), 0) + j * BN
    local_arg = jnp.min(jnp.where(dist == local_min, row, BIG_I32),
                        axis=0, keepdims=True)

    is_start = (j == _WIN_START[0]) | (j == _WIN_START[1]) | (j == _WIN_START[2])

    @pl.when(is_start)
    def _win_init():
        win_v[...] = local_min
        win_i[...] = local_arg

    @pl.when(jnp.logical_not(is_start))
    def _win_merge():
        better = local_min < win_v[...]
        win_v[...] = jnp.where(better, local_min, win_v[...])
        win_i[...] = jnp.where(better, local_arg, win_i[...])

    @pl.when(j == _WIN_END[0])
    def _glob_init():
        glob_v[...] = _bf16(win_v[...])
        glob_i[...] = win_i[...]
        glob_e[...] = win_v[...]

    @pl.when((j == _WIN_END[1]) | (j == _WIN_END[2]))
    def _glob_merge():
        gv, gi = glob_v[...], glob_i[...]
        wv, wi = win_v[...], win_i[...]
        keep = (gv < wv) | ((gv == wv) & (gi < wi))
        glob_i[...] = jnp.where(keep, gi, wi)
        glob_e[...] = jnp.where(keep, glob_e[...], wv)
        glob_v[...] = _bf16(jnp.where(keep, gv, wv))

    @pl.when(j == N_CT - 1)
    def _emit():
        idx_out[...] = glob_i[...].reshape(1, 1, BM)
        min_out[...] = glob_e[...].reshape(1, 1, BM)


def _argmin_tc(table, flat_t, x_sq_t, e_sq_col):
    return pl.pallas_call(
        _dist_body,
        grid=(N_RT, N_CT),
        in_specs=[
            pl.BlockSpec((BN, DIM), lambda i, j: (j, 0)),
            pl.BlockSpec((DIM, BM), lambda i, j: (0, i)),
            pl.BlockSpec((1, BM), lambda i, j: (0, i)),
            pl.BlockSpec((BN, 1), lambda i, j: (j, 0)),
        ],
        out_specs=[
            pl.BlockSpec((1, 1, BM), lambda i, j: (i, 0, 0)),
            pl.BlockSpec((1, 1, BM), lambda i, j: (i, 0, 0)),
        ],
        out_shape=[
            jax.ShapeDtypeStruct((N_RT, 1, BM), jnp.int32),
            jax.ShapeDtypeStruct((N_RT, 1, BM), jnp.float32),
        ],
        scratch_shapes=[
            pltpu.VMEM((1, BM), jnp.float32),
            pltpu.VMEM((1, BM), jnp.int32),
            pltpu.VMEM((1, BM), jnp.float32),
            pltpu.VMEM((1, BM), jnp.int32),
            pltpu.VMEM((1, BM), jnp.float32),
        ],
    )(table, flat_t, x_sq_t, e_sq_col)


_NC = 2
_NS = 16
_NW = _NC * _NS
_BPW = N_ROWS // _NW
_CHUNK = 128
_NCH = _BPW // _CHUNK
_HSLICE = N_CODES // _NS


def _sc_gather_hist(idx2, table):
    mesh = plsc.VectorSubcoreMesh(core_axis_name="c", subcore_axis_name="s")

    @functools.partial(
        pl.kernel,
        out_type=[
            jax.ShapeDtypeStruct((N_ROWS, DIM), jnp.float32),
            jax.ShapeDtypeStruct((_NC, N_CODES), jnp.float32),
        ],
        mesh=mesh,
        scratch_types=[
            pltpu.VMEM((_NCH, _CHUNK), jnp.int32),
            pltpu.VMEM((_CHUNK, DIM), jnp.float32),
            pltpu.VMEM((_CHUNK,), jnp.float32),
            pltpu.VMEM((_HSLICE,), jnp.float32),
            pltpu.VMEM_SHARED((N_CODES,), jnp.float32),
            pltpu.SemaphoreType.DMA,
        ],
    )
    def k(idx_hbm, table_hbm, out_hbm, hist_hbm,
          idx_v, rows_v, ones_v, stage_v, hist_sh, sem):
        cid = lax.axis_index("c")
        sid = lax.axis_index("s")
        wid = sid * _NC + cid
        base = wid * _BPW

        pltpu.sync_copy(idx_hbm.at[wid], idx_v)

        def fill(r, val, n):
            def body(t, _):
                r[pl.ds(t * 16, 16)] = jnp.full((16,), val, jnp.float32)
                return 0
            lax.fori_loop(0, n // 16, body, 0)

        fill(ones_v, 1.0, _CHUNK)
        fill(stage_v, 0.0, _HSLICE)
        pltpu.sync_copy(stage_v, hist_sh.at[pl.ds(sid * _HSLICE, _HSLICE)])
        plsc.subcore_barrier()

        def chunk(c, _):
            pltpu.async_copy(table_hbm.at[idx_v.at[c]], rows_v, sem).wait()
            pltpu.sync_copy(rows_v,
                            out_hbm.at[pl.ds(base + c * _CHUNK, _CHUNK)])
            pltpu.sync_copy(ones_v, hist_sh.at[idx_v.at[c]], add=True)
            return 0
        lax.fori_loop(0, _NCH, chunk, 0)

        plsc.subcore_barrier()
        pltpu.sync_copy(hist_sh.at[pl.ds(sid * _HSLICE, _HSLICE)], stage_v)
        pltpu.sync_copy(stage_v,
                        hist_hbm.at[cid, pl.ds(sid * _HSLICE, _HSLICE)])

    return k(idx2, table)


def _finalize_body(min_ref, hist_ref, loss_ref, perp_ref):
    loss_ref[0, 0] = (COMMIT / (N_ROWS * DIM)) * jnp.sum(min_ref[...])
    counts = hist_ref[0:64, :] + hist_ref[64:128, :]
    p = counts * (1.0 / N_ROWS)
    ent = -jnp.sum(p * jnp.log(p + 1e-10))
    perp_ref[0, 0] = jnp.exp(ent)


def _finalize_tc(min2d, hist2d):
    return pl.pallas_call(
        _finalize_body,
        out_shape=[
            jax.ShapeDtypeStruct((1, 1), jnp.float32),
            jax.ShapeDtypeStruct((1, 1), jnp.float32),
        ],
        out_specs=[
            pl.BlockSpec(memory_space=pltpu.SMEM),
            pl.BlockSpec(memory_space=pltpu.SMEM),
        ],
    )(min2d, hist2d)


def kernel(inputs, embed):
    B, C, H, W = inputs.shape
    flat = jnp.transpose(inputs, (0, 2, 3, 1)).reshape(-1, C)
    table = embed.T

    x_sq_t = jnp.sum(flat ** 2, axis=1, keepdims=True).reshape(1, N_ROWS)
    e_sq_col = jnp.sum(embed ** 2, axis=0).reshape(N_CODES, 1)
    flat_t = jnp.transpose(inputs, (1, 0, 2, 3)).reshape(C, -1)

    idx_col, min_col = _argmin_tc(table, flat_t, x_sq_t, e_sq_col)
    idx_flat = idx_col.reshape(-1)

    idx2 = idx_flat.reshape(_NW, _NCH, _CHUNK)
    quant_flat, hist = _sc_gather_hist(idx2, table)

    loss2d, perp2d = _finalize_tc(min_col.reshape(128, 128),
                                  hist.reshape(128, 128))

    quantized_out = jnp.transpose(quant_flat.reshape(B, H, W, C), (0, 3, 1, 2))
    return (quantized_out,
            loss2d.reshape(()),
            idx_flat,
            perp2d.reshape(()))


# speedup vs baseline: 5.1769x; 1.0414x over previous
import functools

import jax
import jax.numpy as jnp
from jax import lax
from jax.experimental import pallas as pl
from jax.experimental.pallas import tpu as pltpu
from jax.experimental.pallas import tpu_sc as plsc

N_ROWS = 16384
DIM = 256
N_CODES = 8192
COMMIT = 0.25

BM = 8192
BN = 256
N_RT = N_ROWS // BM
N_CT = N_CODES // BN

BIG_I32 = 2**30

_WIN_START = (0, 11, 22)
_WIN_END = (10, 21, 31)


def _bf16(v):
    return v.astype(jnp.bfloat16).astype(jnp.float32)


def _dist_body(et_ref, xt_ref, xsq_ref, esq_ref, idx_out, min_out,
               win_v, win_i, glob_v, glob_i, glob_e):
    j = pl.program_id(1)

    et = et_ref[...]
    xt = xt_ref[...]
    mm = jnp.dot(et, xt, preferred_element_type=jnp.float32)
    dist = (xsq_ref[...] + esq_ref[...]) - 2.0 * mm

    local_min = jnp.min(dist, axis=0, keepdims=True)
    row = jax.lax.broadcasted_iota(jnp.int32, (BN, BM), 0) + j * BN
    local_arg = jnp.min(jnp.where(dist == local_min, row, BIG_I32),
                        axis=0, keepdims=True)

    is_start = (j == _WIN_START[0]) | (j == _WIN_START[1]) | (j == _WIN_START[2])

    @pl.when(is_start)
    def _win_init():
        win_v[...] = local_min
        win_i[...] = local_arg

    @pl.when(jnp.logical_not(is_start))
    def _win_merge():
        better = local_min < win_v[...]
        win_v[...] = jnp.where(better, local_min, win_v[...])
        win_i[...] = jnp.where(better, local_arg, win_i[...])

    @pl.when(j == _WIN_END[0])
    def _glob_init():
        glob_v[...] = _bf16(win_v[...])
        glob_i[...] = win_i[...]
        glob_e[...] = win_v[...]

    @pl.when((j == _WIN_END[1]) | (j == _WIN_END[2]))
    def _glob_merge():
        gv, gi = glob_v[...], glob_i[...]
        wv, wi = win_v[...], win_i[...]
        keep = (gv < wv) | ((gv == wv) & (gi < wi))
        glob_i[...] = jnp.where(keep, gi, wi)
        glob_e[...] = jnp.where(keep, glob_e[...], wv)
        glob_v[...] = _bf16(jnp.where(keep, gv, wv))

    @pl.when(j == N_CT - 1)
    def _emit():
        idx_out[...] = glob_i[...].reshape(1, 1, BM)
        min_out[...] = glob_e[...].reshape(1, 1, BM)


def _argmin_tc(table, flat_t, x_sq_t, e_sq_col):
    return pl.pallas_call(
        _dist_body,
        grid=(N_RT, N_CT),
        in_specs=[
            pl.BlockSpec((BN, DIM), lambda i, j: (j, 0)),
            pl.BlockSpec((DIM, BM), lambda i, j: (0, i)),
            pl.BlockSpec((1, BM), lambda i, j: (0, i)),
            pl.BlockSpec((BN, 1), lambda i, j: (j, 0)),
        ],
        out_specs=[
            pl.BlockSpec((1, 1, BM), lambda i, j: (i, 0, 0)),
            pl.BlockSpec((1, 1, BM), lambda i, j: (i, 0, 0)),
        ],
        out_shape=[
            jax.ShapeDtypeStruct((N_RT, 1, BM), jnp.int32),
            jax.ShapeDtypeStruct((N_RT, 1, BM), jnp.float32),
        ],
        scratch_shapes=[
            pltpu.VMEM((1, BM), jnp.float32),
            pltpu.VMEM((1, BM), jnp.int32),
            pltpu.VMEM((1, BM), jnp.float32),
            pltpu.VMEM((1, BM), jnp.int32),
            pltpu.VMEM((1, BM), jnp.float32),
        ],
    )(table, flat_t, x_sq_t, e_sq_col)


_NC = 2
_NS = 16
_NW = _NC * _NS
_BPW = N_ROWS // _NW
_CHUNK = 128
_NCH = _BPW // _CHUNK
_HSLICE = N_CODES // _NS


def _sc_gather_hist(idx2, table):
    mesh = plsc.VectorSubcoreMesh(core_axis_name="c", subcore_axis_name="s")

    @functools.partial(
        pl.kernel,
        out_type=[
            jax.ShapeDtypeStruct((N_ROWS, DIM), jnp.float32),
            jax.ShapeDtypeStruct((_NC, N_CODES), jnp.float32),
        ],
        mesh=mesh,
        scratch_types=[
            pltpu.VMEM((_NCH, _CHUNK), jnp.int32),
            pltpu.VMEM((_CHUNK, DIM), jnp.float32),
            pltpu.VMEM((_CHUNK,), jnp.float32),
            pltpu.VMEM((_HSLICE,), jnp.float32),
            pltpu.VMEM_SHARED((N_CODES,), jnp.float32),
            pltpu.SemaphoreType.DMA,
        ],
    )
    def k(idx_hbm, table_hbm, out_hbm, hist_hbm,
          idx_v, rows_v, ones_v, stage_v, hist_sh, sem):
        cid = lax.axis_index("c")
        sid = lax.axis_index("s")
        wid = sid * _NC + cid
        base = wid * _BPW

        pltpu.sync_copy(idx_hbm.at[wid], idx_v)

        def fill(r, val, n):
            def body(t, _):
                r[pl.ds(t * 16, 16)] = jnp.full((16,), val, jnp.float32)
                return 0
            lax.fori_loop(0, n // 16, body, 0)

        fill(ones_v, 1.0, _CHUNK)
        fill(stage_v, 0.0, _HSLICE)
        pltpu.sync_copy(stage_v, hist_sh.at[pl.ds(sid * _HSLICE, _HSLICE)])
        plsc.subcore_barrier()

        def chunk(c, _):
            pltpu.async_copy(table_hbm.at[idx_v.at[c]], rows_v, sem).wait()
            pltpu.sync_copy(rows_v,
                            out_hbm.at[pl.ds(base + c * _CHUNK, _CHUNK)])
            pltpu.sync_copy(ones_v, hist_sh.at[idx_v.at[c]], add=True)
            return 0
        lax.fori_loop(0, _NCH, chunk, 0)

        plsc.subcore_barrier()
        pltpu.sync_copy(hist_sh.at[pl.ds(sid * _HSLICE, _HSLICE)], stage_v)
        pltpu.sync_copy(stage_v,
                        hist_hbm.at[cid, pl.ds(sid * _HSLICE, _HSLICE)])

    return k(idx2, table)


def _finalize_body(min_ref, hist_ref, loss_ref, perp_ref):
    loss_ref[0, 0] = (COMMIT / (N_ROWS * DIM)) * jnp.sum(min_ref[...])
    counts = hist_ref[0:64, :] + hist_ref[64:128, :]
    p = counts * (1.0 / N_ROWS)
    ent = -jnp.sum(p * jnp.log(p + 1e-10))
    perp_ref[0, 0] = jnp.exp(ent)


def _finalize_tc(min2d, hist2d):
    return pl.pallas_call(
        _finalize_body,
        out_shape=[
            jax.ShapeDtypeStruct((1, 1), jnp.float32),
            jax.ShapeDtypeStruct((1, 1), jnp.float32),
        ],
        out_specs=[
            pl.BlockSpec(memory_space=pltpu.SMEM),
            pl.BlockSpec(memory_space=pltpu.SMEM),
        ],
    )(min2d, hist2d)


def kernel(inputs, embed):
    B, C, H, W = inputs.shape
    flat = jnp.transpose(inputs, (0, 2, 3, 1)).reshape(-1, C)
    table = embed.T

    x_sq_t = jnp.sum(flat ** 2, axis=1, keepdims=True).reshape(1, N_ROWS)
    e_sq_col = jnp.sum(embed ** 2, axis=0).reshape(N_CODES, 1)
    flat_t = jnp.transpose(inputs, (1, 0, 2, 3)).reshape(C, -1)

    idx_col, min_col = _argmin_tc(table, flat_t, x_sq_t, e_sq_col)
    idx_flat = idx_col.reshape(-1)

    idx2 = idx_flat.reshape(_NW, _NCH, _CHUNK)
    quant_flat, hist = _sc_gather_hist(idx2, table)

    loss2d, perp2d = _finalize_tc(min_col.reshape(128, 128),
                                  hist.reshape(128, 128))

    quantized_out = jnp.transpose(quant_flat.reshape(B, H, W, C), (0, 3, 1, 2))
    return (quantized_out,
            loss2d.reshape(()),
            idx_flat,
            perp2d.reshape(()))


# speedup vs baseline: 5.2361x; 1.0114x over previous
import functools

import jax
import jax.numpy as jnp
from jax import lax
from jax.experimental import pallas as pl
from jax.experimental.pallas import tpu as pltpu
from jax.experimental.pallas import tpu_sc as plsc

N_ROWS = 16384
DIM = 256
N_CODES = 8192
COMMIT = 0.25

BM = 16384
BN = 256
N_RT = N_ROWS // BM
N_CT = N_CODES // BN

BIG_I32 = 2**30

_WIN_START = (0, 11, 22)
_WIN_END = (10, 21, 31)


def _bf16(v):
    return v.astype(jnp.bfloat16).astype(jnp.float32)


def _dist_body(et_ref, xt_ref, xsq_ref, esq_ref, idx_out, min_out,
               win_v, win_i, glob_v, glob_i, glob_e):
    j = pl.program_id(1)

    et = et_ref[...]
    xt = xt_ref[...]
    mm = jnp.dot(et, xt, preferred_element_type=jnp.float32)
    dist = (xsq_ref[...] + esq_ref[...]) - 2.0 * mm

    local_min = jnp.min(dist, axis=0, keepdims=True)
    row = jax.lax.broadcasted_iota(jnp.int32, (BN, BM), 0) + j * BN
    local_arg = jnp.min(jnp.where(dist == local_min, row, BIG_I32),
                        axis=0, keepdims=True)

    is_start = (j == _WIN_START[0]) | (j == _WIN_START[1]) | (j == _WIN_START[2])

    @pl.when(is_start)
    def _win_init():
        win_v[...] = local_min
        win_i[...] = local_arg

    @pl.when(jnp.logical_not(is_start))
    def _win_merge():
        better = local_min < win_v[...]
        win_v[...] = jnp.where(better, local_min, win_v[...])
        win_i[...] = jnp.where(better, local_arg, win_i[...])

    @pl.when(j == _WIN_END[0])
    def _glob_init():
        glob_v[...] = _bf16(win_v[...])
        glob_i[...] = win_i[...]
        glob_e[...] = win_v[...]

    @pl.when((j == _WIN_END[1]) | (j == _WIN_END[2]))
    def _glob_merge():
        gv, gi = glob_v[...], glob_i[...]
        wv, wi = win_v[...], win_i[...]
        keep = (gv < wv) | ((gv == wv) & (gi < wi))
        glob_i[...] = jnp.where(keep, gi, wi)
        glob_e[...] = jnp.where(keep, glob_e[...], wv)
        glob_v[...] = _bf16(jnp.where(keep, gv, wv))

    @pl.when(j == N_CT - 1)
    def _emit():
        idx_out[...] = glob_i[...].reshape(1, 1, BM)
        min_out[...] = glob_e[...].reshape(1, 1, BM)


def _argmin_tc(table, flat_t, x_sq_t, e_sq_col):
    return pl.pallas_call(
        _dist_body,
        grid=(N_RT, N_CT),
        in_specs=[
            pl.BlockSpec((BN, DIM), lambda i, j: (j, 0)),
            pl.BlockSpec((DIM, BM), lambda i, j: (0, i)),
            pl.BlockSpec((1, BM), lambda i, j: (0, i)),
            pl.BlockSpec((BN, 1), lambda i, j: (j, 0)),
        ],
        out_specs=[
            pl.BlockSpec((1, 1, BM), lambda i, j: (i, 0, 0)),
            pl.BlockSpec((1, 1, BM), lambda i, j: (i, 0, 0)),
        ],
        out_shape=[
            jax.ShapeDtypeStruct((N_RT, 1, BM), jnp.int32),
            jax.ShapeDtypeStruct((N_RT, 1, BM), jnp.float32),
        ],
        scratch_shapes=[
            pltpu.VMEM((1, BM), jnp.float32),
            pltpu.VMEM((1, BM), jnp.int32),
            pltpu.VMEM((1, BM), jnp.float32),
            pltpu.VMEM((1, BM), jnp.int32),
            pltpu.VMEM((1, BM), jnp.float32),
        ],
    )(table, flat_t, x_sq_t, e_sq_col)


_NC = 2
_NS = 16
_NW = _NC * _NS
_BPW = N_ROWS // _NW
_CHUNK = 128
_NCH = _BPW // _CHUNK
_HSLICE = N_CODES // _NS


def _sc_gather_hist(idx2, table):
    mesh = plsc.VectorSubcoreMesh(core_axis_name="c", subcore_axis_name="s")

    @functools.partial(
        pl.kernel,
        out_type=[
            jax.ShapeDtypeStruct((N_ROWS, DIM), jnp.float32),
            jax.ShapeDtypeStruct((_NC, N_CODES), jnp.float32),
        ],
        mesh=mesh,
        scratch_types=[
            pltpu.VMEM((_NCH, _CHUNK), jnp.int32),
            pltpu.VMEM((_CHUNK, DIM), jnp.float32),
            pltpu.VMEM((_CHUNK,), jnp.float32),
            pltpu.VMEM((_HSLICE,), jnp.float32),
            pltpu.VMEM_SHARED((N_CODES,), jnp.float32),
            pltpu.SemaphoreType.DMA,
        ],
    )
    def k(idx_hbm, table_hbm, out_hbm, hist_hbm,
          idx_v, rows_v, ones_v, stage_v, hist_sh, sem):
        cid = lax.axis_index("c")
        sid = lax.axis_index("s")
        wid = sid * _NC + cid
        base = wid * _BPW

        pltpu.sync_copy(idx_hbm.at[wid], idx_v)

        def fill(r, val, n):
            def body(t, _):
                r[pl.ds(t * 16, 16)] = jnp.full((16,), val, jnp.float32)
                return 0
            lax.fori_loop(0, n // 16, body, 0)

        fill(ones_v, 1.0, _CHUNK)
        fill(stage_v, 0.0, _HSLICE)
        pltpu.sync_copy(stage_v, hist_sh.at[pl.ds(sid * _HSLICE, _HSLICE)])
        plsc.subcore_barrier()

        def chunk(c, _):
            pltpu.async_copy(table_hbm.at[idx_v.at[c]], rows_v, sem).wait()
            pltpu.sync_copy(rows_v,
                            out_hbm.at[pl.ds(base + c * _CHUNK, _CHUNK)])
            pltpu.sync_copy(ones_v, hist_sh.at[idx_v.at[c]], add=True)
            return 0
        lax.fori_loop(0, _NCH, chunk, 0)

        plsc.subcore_barrier()
        pltpu.sync_copy(hist_sh.at[pl.ds(sid * _HSLICE, _HSLICE)], stage_v)
        pltpu.sync_copy(stage_v,
                        hist_hbm.at[cid, pl.ds(sid * _HSLICE, _HSLICE)])

    return k(idx2, table)


def _finalize_body(min_ref, hist_ref, loss_ref, perp_ref):
    loss_ref[0, 0] = (COMMIT / (N_ROWS * DIM)) * jnp.sum(min_ref[...])
    counts = hist_ref[0:64, :] + hist_ref[64:128, :]
    p = counts * (1.0 / N_ROWS)
    ent = -jnp.sum(p * jnp.log(p + 1e-10))
    perp_ref[0, 0] = jnp.exp(ent)


def _finalize_tc(min2d, hist2d):
    return pl.pallas_call(
        _finalize_body,
        out_shape=[
            jax.ShapeDtypeStruct((1, 1), jnp.float32),
            jax.ShapeDtypeStruct((1, 1), jnp.float32),
        ],
        out_specs=[
            pl.BlockSpec(memory_space=pltpu.SMEM),
            pl.BlockSpec(memory_space=pltpu.SMEM),
        ],
    )(min2d, hist2d)


def kernel(inputs, embed):
    B, C, H, W = inputs.shape
    flat = jnp.transpose(inputs, (0, 2, 3, 1)).reshape(-1, C)
    table = embed.T

    x_sq_t = jnp.sum(flat ** 2, axis=1, keepdims=True).reshape(1, N_ROWS)
    e_sq_col = jnp.sum(embed ** 2, axis=0).reshape(N_CODES, 1)
    flat_t = jnp.transpose(inputs, (1, 0, 2, 3)).reshape(C, -1)

    idx_col, min_col = _argmin_tc(table, flat_t, x_sq_t, e_sq_col)
    idx_flat = idx_col.reshape(-1)

    idx2 = idx_flat.reshape(_NW, _NCH, _CHUNK)
    quant_flat, hist = _sc_gather_hist(idx2, table)

    loss2d, perp2d = _finalize_tc(min_col.reshape(128, 128),
                                  hist.reshape(128, 128))

    quantized_out = jnp.transpose(quant_flat.reshape(B, H, W, C), (0, 3, 1, 2))
    return (quantized_out,
            loss2d.reshape(()),
            idx_flat,
            perp2d.reshape(()))


# speedup vs baseline: 5.2850x; 1.0093x over previous
import functools

import jax
import jax.numpy as jnp
from jax import lax
from jax.experimental import pallas as pl
from jax.experimental.pallas import tpu as pltpu
from jax.experimental.pallas import tpu_sc as plsc

N_ROWS = 16384
DIM = 256
N_CODES = 8192
COMMIT = 0.25

BM = 16384
BN = 256
N_RT = N_ROWS // BM
N_CT = N_CODES // BN

BIG_I32 = 2**30

_WIN_START = (0, 11, 22)
_WIN_END = (10, 21, 31)


def _bf16(v):
    return v.astype(jnp.bfloat16).astype(jnp.float32)


def _dist_body(et_ref, xt_ref, xsq_ref, esq_ref, idx_out, min_out,
               win_v, win_i, glob_v, glob_i, glob_e):
    j = pl.program_id(1)

    et = et_ref[...]
    xt = xt_ref[...]
    mm = jnp.dot(et, xt, preferred_element_type=jnp.float32)
    dist = (xsq_ref[...] + esq_ref[...]) - 2.0 * mm

    local_min = jnp.min(dist, axis=0, keepdims=True)
    row = jax.lax.broadcasted_iota(jnp.int32, (BN, BM), 0) + j * BN
    local_arg = jnp.min(jnp.where(dist == local_min, row, BIG_I32),
                        axis=0, keepdims=True)

    is_start = (j == _WIN_START[0]) | (j == _WIN_START[1]) | (j == _WIN_START[2])

    @pl.when(is_start)
    def _win_init():
        win_v[...] = local_min
        win_i[...] = local_arg

    @pl.when(jnp.logical_not(is_start))
    def _win_merge():
        better = local_min < win_v[...]
        win_v[...] = jnp.where(better, local_min, win_v[...])
        win_i[...] = jnp.where(better, local_arg, win_i[...])

    @pl.when(j == _WIN_END[0])
    def _glob_init():
        glob_v[...] = _bf16(win_v[...])
        glob_i[...] = win_i[...]
        glob_e[...] = win_v[...]

    @pl.when((j == _WIN_END[1]) | (j == _WIN_END[2]))
    def _glob_merge():
        gv, gi = glob_v[...], glob_i[...]
        wv, wi = win_v[...], win_i[...]
        keep = (gv < wv) | ((gv == wv) & (gi < wi))
        glob_i[...] = jnp.where(keep, gi, wi)
        glob_e[...] = jnp.where(keep, glob_e[...], wv)
        glob_v[...] = _bf16(jnp.where(keep, gv, wv))

    @pl.when(j == N_CT - 1)
    def _emit():
        idx_out[...] = glob_i[...].reshape(1, 1, BM)
        min_out[...] = glob_e[...].reshape(1, 1, BM)


def _argmin_tc(table, flat_t, x_sq_t, e_sq_col):
    return pl.pallas_call(
        _dist_body,
        grid=(N_RT, N_CT),
        in_specs=[
            pl.BlockSpec((BN, DIM), lambda i, j: (j, 0)),
            pl.BlockSpec((DIM, BM), lambda i, j: (0, i)),
            pl.BlockSpec((1, BM), lambda i, j: (0, i)),
            pl.BlockSpec((BN, 1), lambda i, j: (j, 0)),
        ],
        out_specs=[
            pl.BlockSpec((1, 1, BM), lambda i, j: (i, 0, 0)),
            pl.BlockSpec((1, 1, BM), lambda i, j: (i, 0, 0)),
        ],
        out_shape=[
            jax.ShapeDtypeStruct((N_RT, 1, BM), jnp.int32),
            jax.ShapeDtypeStruct((N_RT, 1, BM), jnp.float32),
        ],
        scratch_shapes=[
            pltpu.VMEM((1, BM), jnp.float32),
            pltpu.VMEM((1, BM), jnp.int32),
            pltpu.VMEM((1, BM), jnp.float32),
            pltpu.VMEM((1, BM), jnp.int32),
            pltpu.VMEM((1, BM), jnp.float32),
        ],
    )(table, flat_t, x_sq_t, e_sq_col)


_NC = 2
_NS = 16
_NW = _NC * _NS
_BPW = N_ROWS // _NW
_CHUNK = 128
_NCH = _BPW // _CHUNK
_HSLICE = N_CODES // _NS


def _sc_gather_hist(idx2, table):
    mesh = plsc.VectorSubcoreMesh(core_axis_name="c", subcore_axis_name="s")

    @functools.partial(
        pl.kernel,
        out_type=[
            jax.ShapeDtypeStruct((N_ROWS, DIM), jnp.float32),
            jax.ShapeDtypeStruct((_NC, N_CODES), jnp.float32),
        ],
        mesh=mesh,
        scratch_types=[
            pltpu.VMEM((_NCH, _CHUNK), jnp.int32),
            pltpu.VMEM((_CHUNK, DIM), jnp.float32),
            pltpu.VMEM((_CHUNK, DIM), jnp.float32),
            pltpu.VMEM((_CHUNK,), jnp.float32),
            pltpu.VMEM((_HSLICE,), jnp.float32),
            pltpu.VMEM_SHARED((N_CODES,), jnp.float32),
            pltpu.SemaphoreType.DMA,
            pltpu.SemaphoreType.DMA,
        ],
    )
    def k(idx_hbm, table_hbm, out_hbm, hist_hbm,
          idx_v, rows_v0, rows_v1, ones_v, stage_v, hist_sh, sem0, sem1):
        cid = lax.axis_index("c")
        sid = lax.axis_index("s")
        wid = sid * _NC + cid
        base = wid * _BPW

        pltpu.sync_copy(idx_hbm.at[wid], idx_v)

        def fill(r, val, n):
            def body(t, _):
                r[pl.ds(t * 16, 16)] = jnp.full((16,), val, jnp.float32)
                return 0
            lax.fori_loop(0, n // 16, body, 0)

        fill(ones_v, 1.0, _CHUNK)
        fill(stage_v, 0.0, _HSLICE)
        pltpu.sync_copy(stage_v, hist_sh.at[pl.ds(sid * _HSLICE, _HSLICE)])
        plsc.subcore_barrier()

        bufs = (rows_v0, rows_v1)
        sems = (sem0, sem1)
        copies = [pltpu.async_copy(table_hbm.at[idx_v.at[c]],
                                   bufs[c % 2], sems[c % 2])
                  for c in range(2)]
        for c in range(_NCH):
            copies[c].wait()
            pltpu.sync_copy(bufs[c % 2],
                            out_hbm.at[pl.ds(base + c * _CHUNK, _CHUNK)])
            if c + 2 < _NCH:
                copies.append(pltpu.async_copy(table_hbm.at[idx_v.at[c + 2]],
                                               bufs[c % 2], sems[c % 2]))
            pltpu.sync_copy(ones_v, hist_sh.at[idx_v.at[c]], add=True)

        plsc.subcore_barrier()
        pltpu.sync_copy(hist_sh.at[pl.ds(sid * _HSLICE, _HSLICE)], stage_v)
        pltpu.sync_copy(stage_v,
                        hist_hbm.at[cid, pl.ds(sid * _HSLICE, _HSLICE)])

    return k(idx2, table)


def _finalize_body(min_ref, hist_ref, loss_ref, perp_ref):
    loss_ref[0, 0] = (COMMIT / (N_ROWS * DIM)) * jnp.sum(min_ref[...])
    counts = hist_ref[0:64, :] + hist_ref[64:128, :]
    p = counts * (1.0 / N_ROWS)
    ent = -jnp.sum(p * jnp.log(p + 1e-10))
    perp_ref[0, 0] = jnp.exp(ent)


def _finalize_tc(min2d, hist2d):
    return pl.pallas_call(
        _finalize_body,
        out_shape=[
            jax.ShapeDtypeStruct((1, 1), jnp.float32),
            jax.ShapeDtypeStruct((1, 1), jnp.float32),
        ],
        out_specs=[
            pl.BlockSpec(memory_space=pltpu.SMEM),
            pl.BlockSpec(memory_space=pltpu.SMEM),
        ],
    )(min2d, hist2d)


def kernel(inputs, embed):
    B, C, H, W = inputs.shape
    flat = jnp.transpose(inputs, (0, 2, 3, 1)).reshape(-1, C)
    table = embed.T

    x_sq_t = jnp.sum(flat ** 2, axis=1, keepdims=True).reshape(1, N_ROWS)
    e_sq_col = jnp.sum(embed ** 2, axis=0).reshape(N_CODES, 1)
    flat_t = jnp.transpose(inputs, (1, 0, 2, 3)).reshape(C, -1)

    idx_col, min_col = _argmin_tc(table, flat_t, x_sq_t, e_sq_col)
    idx_flat = idx_col.reshape(-1)

    idx2 = idx_flat.reshape(_NW, _NCH, _CHUNK)
    quant_flat, hist = _sc_gather_hist(idx2, table)

    loss2d, perp2d = _finalize_tc(min_col.reshape(128, 128),
                                  hist.reshape(128, 128))

    quantized_out = jnp.transpose(quant_flat.reshape(B, H, W, C), (0, 3, 1, 2))
    return (quantized_out,
            loss2d.reshape(()),
            idx_flat,
            perp2d.reshape(()))
